# baseline (speedup 1.0000x reference)
import functools

import jax
import jax.numpy as jnp
from jax import lax
from jax.experimental import pallas as pl
from jax.experimental.pallas import tpu as pltpu
from jax.experimental.pallas import tpu_sc as plsc

N = 10000
E = 320000
D_IN = 128
D_H = 128
D_Z = 64

NC = 2
NS = 16
NW = NC * NS
L = 16

CH = 128
E_P = 327680
NCHP = E_P // CH
NCH_T = NCHP // NW
NGRP = NCH_T // 8
NPAIR = NCH_T // 2
NPAD = 10240
ROWS_PER_TILE = NPAD // NS

_MESH = plsc.VectorSubcoreMesh(core_axis_name="c", subcore_axis_name="s",
                               num_cores=2, num_subcores=16)


def _wid():
    return lax.axis_index("c") * NS + lax.axis_index("s")


@functools.partial(
    pl.kernel,
    out_type=jax.ShapeDtypeStruct((NC, NPAD, L), jnp.float32),
    mesh=_MESH,
    scratch_types=[
        pltpu.VMEM((NCH_T, CH), jnp.int32),
        pltpu.VMEM((CH, L), jnp.float32),
        pltpu.VMEM((CH, L), jnp.float32),
        pltpu.VMEM_SHARED((NPAD, L), jnp.float32),
        pltpu.SemaphoreType.DMA,
    ],
)
def _deg_sc(dst_hbm, deg_hbm, idx_all, ones_v, zb_v, acc, sem):
    cid = lax.axis_index("c")
    sid = lax.axis_index("s")
    wid = _wid()

    def fill(r, _):
        ones_v[r, :] = jnp.full((L,), 1.0, jnp.float32)
        zb_v[r, :] = jnp.zeros((L,), jnp.float32)
        return 0

    lax.fori_loop(0, CH, fill, 0)
    for k in range(NGRP):
        pltpu.sync_copy(dst_hbm.at[pl.ds((k * NW + wid) * 8, 8)],
                        idx_all.at[pl.ds(k * 8, 8)])
    for k in range(5):
        pltpu.sync_copy(
            zb_v, acc.at[pl.ds(sid * ROWS_PER_TILE + k * CH, CH)])
    plsc.subcore_barrier()

    def group(k, _):
        descs = []
        for j in range(8):
            descs.append(
                pltpu.async_copy(ones_v, acc.at[idx_all.at[k * 8 + j]], sem,
                                 add=True))
        for d in descs:
            d.wait()
        return 0

    lax.fori_loop(0, NGRP, group, 0)
    plsc.subcore_barrier()
    pltpu.sync_copy(
        acc.at[pl.ds(sid * ROWS_PER_TILE, ROWS_PER_TILE)],
        deg_hbm.at[cid, pl.ds(sid * ROWS_PER_TILE, ROWS_PER_TILE)],
    )


@functools.partial(
    pl.kernel,
    out_type=jax.ShapeDtypeStruct((NC, NPAD, D_H), jnp.float32),
    mesh=_MESH,
    scratch_types=[
        pltpu.VMEM((CH,), jnp.int32),
        pltpu.VMEM((CH,), jnp.int32),
        pltpu.VMEM((CH,), jnp.int32),
        pltpu.VMEM((CH,), jnp.int32),
        pltpu.VMEM((CH, D_H), jnp.float32),
        pltpu.VMEM((CH, D_H), jnp.float32),
        pltpu.VMEM_SHARED((NPAD, D_H), jnp.float32),
        pltpu.SemaphoreType.DMA,
        pltpu.SemaphoreType.DMA,
        pltpu.SemaphoreType.DMA,
        pltpu.SemaphoreType.DMA,
        pltpu.SemaphoreType.DMA,
        pltpu.SemaphoreType.DMA,
    ],
)
def _agg_sc(g_hbm, src_hbm, dst_hbm, out_hbm, is0, is1, id0, id1, rows0,
            rows1, acc, gs0, gs1, es0, es1, fs0, fs1):
    cid = lax.axis_index("c")
    sid = lax.axis_index("s")
    wid = _wid()
    rows = (rows0, rows1)
    iss = (is0, is1)
    ids = (id0, id1)
    gsem = (gs0, gs1)
    esem = (es0, es1)
    fsem = (fs0, fs1)

    def row(c):
        return c * NW + wid

    def fill(r, _):
        for c8 in range(D_H // L):
            rows0[r, pl.ds(c8 * L, L)] = jnp.zeros((L,), jnp.float32)
        return 0

    lax.fori_loop(0, CH, fill, 0)
    for k in range(5):
        pltpu.sync_copy(
            rows0, acc.at[pl.ds(sid * ROWS_PER_TILE + k * CH, CH)])
    plsc.subcore_barrier()

    pltpu.sync_copy(src_hbm.at[row(0)], is0)
    pltpu.sync_copy(dst_hbm.at[row(0)], id0)
    pltpu.async_copy(src_hbm.at[row(1)], is1, es1)
    pltpu.async_copy(dst_hbm.at[row(1)], id1, fs1)
    pltpu.async_copy(g_hbm.at[is0], rows0, gs0)

    def pair(p, _):
        for b in range(2):
            c = 2 * p + b
            pltpu.make_async_copy(g_hbm.at[iss[b]], rows[b], gsem[b]).wait()
            pltpu.sync_copy(rows[b], acc.at[ids[b]], add=True)

            @pl.when(c + 2 < NCH_T)
            def _():
                pltpu.async_copy(src_hbm.at[row(c + 2)], iss[b], esem[b])
                pltpu.async_copy(dst_hbm.at[row(c + 2)], ids[b], fsem[b])

            @pl.when(c + 1 < NCH_T)
            def _():
                pltpu.make_async_copy(src_hbm.at[row(c + 1)], iss[1 - b],
                                      esem[1 - b]).wait()
                pltpu.make_async_copy(dst_hbm.at[row(c + 1)], ids[1 - b],
                                      fsem[1 - b]).wait()
                pltpu.async_copy(g_hbm.at[iss[1 - b]], rows[1 - b],
                                 gsem[1 - b])
        return 0

    lax.fori_loop(0, NPAIR, pair, 0)
    plsc.subcore_barrier()
    pltpu.sync_copy(
        acc.at[pl.ds(sid * ROWS_PER_TILE, ROWS_PER_TILE)],
        out_hbm.at[cid, pl.ds(sid * ROWS_PER_TILE, ROWS_PER_TILE)],
    )


@functools.partial(
    pl.kernel,
    out_type=jax.ShapeDtypeStruct((E_P * L,), jnp.float32),
    mesh=_MESH,
    scratch_types=[
        pltpu.VMEM((NCH_T, CH), jnp.int32),
        pltpu.VMEM((NCH_T, CH), jnp.int32),
        pltpu.VMEM((CH, D_H), jnp.float32),
        pltpu.VMEM((CH, D_H), jnp.float32),
        pltpu.VMEM((CH, D_H), jnp.float32),
        pltpu.VMEM((CH, D_H), jnp.float32),
        pltpu.VMEM((CH * L,), jnp.float32),
        pltpu.VMEM((CH * L,), jnp.float32),
        pltpu.SemaphoreType.DMA,
        pltpu.SemaphoreType.DMA,
        pltpu.SemaphoreType.DMA,
        pltpu.SemaphoreType.DMA,
        pltpu.SemaphoreType.DMA,
        pltpu.SemaphoreType.DMA,
    ],
)
def _dec_sc(z_hbm, src_hbm, dst_hbm, q_hbm, idx_s, idx_d, zs0, zs1, zd0, zd1,
            q0, q1, ss0, ss1, sd0, sd1, sq0, sq1):
    wid = _wid()
    zs = (zs0, zs1)
    zd = (zd0, zd1)
    qb = (q0, q1)
    sss = (ss0, ss1)
    sds = (sd0, sd1)
    sqs = (sq0, sq1)

    for k in range(NGRP):
        pltpu.sync_copy(src_hbm.at[pl.ds((k * NW + wid) * 8, 8)],
                        idx_s.at[pl.ds(k * 8, 8)])
        pltpu.sync_copy(dst_hbm.at[pl.ds((k * NW + wid) * 8, 8)],
                        idx_d.at[pl.ds(k * 8, 8)])

    pltpu.async_copy(z_hbm.at[idx_s.at[0]], zs0, ss0)
    pltpu.async_copy(z_hbm.at[idx_d.at[0]], zd0, sd0)

    def pair(p, _):
        for b in range(2):
            c = 2 * p + b
            base = (c * NW + wid) * CH

            @pl.when(c + 1 < NCH_T)
            def _():
                pltpu.async_copy(z_hbm.at[idx_s.at[c + 1]], zs[1 - b],
                                 sss[1 - b])
                pltpu.async_copy(z_hbm.at[idx_d.at[c + 1]], zd[1 - b],
                                 sds[1 - b])

            pltpu.make_async_copy(z_hbm.at[idx_s.at[c]], zs[b], sss[b]).wait()
            pltpu.make_async_copy(z_hbm.at[idx_d.at[c]], zd[b], sds[b]).wait()

            @pl.when(c >= 2)
            def _():
                pltpu.make_async_copy(
                    qb[b], q_hbm.at[pl.ds(base * L, CH * L)], sqs[b]).wait()

            zsb = zs[b]
            zdb = zd[b]
            qvb = qb[b]

            def dot_edge(i, _):
                for u in range(2):
                    e = 2 * i + u
                    q = zsb[e, pl.ds(0, L)] * zdb[e, pl.ds(0, L)]
                    for s in range(1, D_Z // L):
                        q = q + (zsb[e, pl.ds(s * L, L)] *
                                 zdb[e, pl.ds(s * L, L)])
                    qvb[pl.ds(e * L, L)] = q
                return 0

            lax.fori_loop(0, CH // 2, dot_edge, 0)
            pltpu.async_copy(qvb, q_hbm.at[pl.ds(base * L, CH * L)], sqs[b])
        return 0

    lax.fori_loop(0, NPAIR, pair, 0)
    for b in range(2):
        c = NCH_T - 2 + b
        base = (c * NW + wid) * CH
        pltpu.make_async_copy(
            qb[b], q_hbm.at[pl.ds(base * L, CH * L)], sqs[b]).wait()


def _t1_body(x_ref, w_ref, d0_ref, d1_ref, g_ref):
    deg = d0_ref[:, 0:1] + d1_ref[:, 0:1] + 1.0
    dinv = lax.rsqrt(jnp.maximum(deg, 1e-12))
    h = jnp.dot(x_ref[...], w_ref[...], preferred_element_type=jnp.float32)
    g_ref[...] = h * dinv


def _t2_body(s0_ref, s1_ref, g1_ref, d0_ref, d1_ref, b1_ref, w_ref, g2_ref):
    deg = d0_ref[:, 0:1] + d1_ref[:, 0:1] + 1.0
    dinv = lax.rsqrt(jnp.maximum(deg, 1e-12))
    h = jnp.maximum(
        dinv * (s0_ref[...] + s1_ref[...] + g1_ref[...]) + b1_ref[...], 0.0)
    p = jnp.dot(h, w_ref[...], preferred_element_type=jnp.float32)
    g2_ref[...] = p * dinv


def _t3_body(s0_ref, s1_ref, g2_ref, d0_ref, d1_ref, bc_ref, eps_ref, z_ref):
    deg = d0_ref[:, 0:1] + d1_ref[:, 0:1] + 1.0
    dinv = lax.rsqrt(jnp.maximum(deg, 1e-12))
    o = dinv * (s0_ref[...] + s1_ref[...] + g2_ref[...]) + bc_ref[...]
    mu = o[:, :D_Z]
    lv = o[:, D_Z:]
    z = mu + jnp.exp(0.5 * lv) * eps_ref[...]
    z_ref[...] = jnp.concatenate([z, jnp.zeros_like(z)], axis=1)


def _t4_body(q_ref, o_ref):
    o_ref[...] = jax.nn.sigmoid(jnp.sum(q_ref[...], axis=1, keepdims=True))


_RB = 1000
_GRID = N // _RB


def _row_spec(width):
    return pl.BlockSpec((_RB, width), lambda i: (i, 0))


def _full_spec(shape):
    return pl.BlockSpec(shape, lambda i: tuple(0 for _ in shape))


def kernel(x, edge_index, W1, b1, W_mu, b_mu, W_lv, b_lv):
    src = edge_index[0]
    dst = edge_index[1]
    pad = E_P - E
    src_a = jnp.concatenate([src, jnp.zeros((pad,), src.dtype)])
    dst_a = jnp.concatenate([dst, jnp.full((pad,), NPAD - 1, dst.dtype)])
    dst_0 = jnp.concatenate([dst, jnp.zeros((pad,), dst.dtype)])
    src2 = src_a.reshape(NCHP, CH)
    dst2 = dst_a.reshape(NCHP, CH)
    dst2_0 = dst_0.reshape(NCHP, CH)
    Wcat = jnp.concatenate([W_mu, W_lv], axis=1)
    bcat = jnp.concatenate([b_mu, b_lv], axis=0).reshape(1, 2 * D_Z)
    b1r = b1.reshape(1, D_H)
    eps = jax.random.normal(jax.random.key(42), (N, D_Z), jnp.float32)

    deg_parts = _deg_sc(dst2)
    d0 = deg_parts[0, :N]
    d1 = deg_parts[1, :N]

    g1 = pl.pallas_call(
        _t1_body,
        grid=(_GRID,),
        in_specs=[_row_spec(D_IN), _full_spec((D_IN, D_H)), _row_spec(L),
                  _row_spec(L)],
        out_specs=_row_spec(D_H),
        out_shape=jax.ShapeDtypeStruct((N, D_H), jnp.float32),
    )(x, W1, d0, d1)

    s1p = _agg_sc(g1, src2, dst2)
    s1 = (s1p[0, :N], s1p[1, :N])

    g2 = pl.pallas_call(
        _t2_body,
        grid=(_GRID,),
        in_specs=[_row_spec(D_H), _row_spec(D_H), _row_spec(D_H),
                  _row_spec(L), _row_spec(L), _full_spec((1, D_H)),
                  _full_spec((D_H, D_H))],
        out_specs=_row_spec(D_H),
        out_shape=jax.ShapeDtypeStruct((N, D_H), jnp.float32),
    )(s1[0], s1[1], g1, d0, d1, b1r, Wcat)

    s2p = _agg_sc(g2, src2, dst2)
    s2 = (s2p[0, :N], s2p[1, :N])

    z = pl.pallas_call(
        _t3_body,
        grid=(_GRID,),
        in_specs=[_row_spec(D_H), _row_spec(D_H), _row_spec(D_H),
                  _row_spec(L), _row_spec(L), _full_spec((1, D_H)),
                  _row_spec(D_Z)],
        out_specs=_row_spec(D_H),
        out_shape=jax.ShapeDtypeStruct((N, D_H), jnp.float32),
    )(s2[0], s2[1], g2, d0, d1, bcat, eps)

    qflat = _dec_sc(z, src2, dst2_0)
    q = qflat.reshape(E_P, L)

    _EB = 4096
    out = pl.pallas_call(
        _t4_body,
        grid=(E_P // _EB,),
        in_specs=[pl.BlockSpec((_EB, L), lambda i: (i, 0))],
        out_specs=pl.BlockSpec((_EB, 1), lambda i: (i, 0)),
        out_shape=jax.ShapeDtypeStruct((E_P, 1), jnp.float32),
    )(q)
    return out[:E].reshape(E)


# speedup vs baseline: 1.0436x; 1.0436x over previous
import functools

import jax
import jax.numpy as jnp
from jax import lax
from jax.experimental import pallas as pl
from jax.experimental.pallas import tpu as pltpu
from jax.experimental.pallas import tpu_sc as plsc

N = 10000
E = 320000
D_IN = 128
D_H = 128
D_Z = 64

NC = 2
NS = 16
NW = NC * NS
L = 16

CH = 128
E_P = 327680
NCHP = E_P // CH
NCH_T = NCHP // NW
NGRP = NCH_T // 8
NPAIR = NCH_T // 2
NPAD = 10240
ROWS_PER_TILE = NPAD // NS

_MESH = plsc.VectorSubcoreMesh(core_axis_name="c", subcore_axis_name="s",
                               num_cores=2, num_subcores=16)


def _wid():
    return lax.axis_index("c") * NS + lax.axis_index("s")


@functools.partial(
    pl.kernel,
    out_type=jax.ShapeDtypeStruct((NC, NPAD, L), jnp.float32),
    mesh=_MESH,
    scratch_types=[
        pltpu.VMEM((NCH_T, CH), jnp.int32),
        pltpu.VMEM((CH, L), jnp.float32),
        pltpu.VMEM((CH, L), jnp.float32),
        pltpu.VMEM_SHARED((NPAD, L), jnp.float32),
        pltpu.SemaphoreType.DMA,
    ],
)
def _deg_sc(dst_hbm, deg_hbm, idx_all, ones_v, zb_v, acc, sem):
    cid = lax.axis_index("c")
    sid = lax.axis_index("s")
    wid = _wid()

    def fill(r, _):
        ones_v[r, :] = jnp.full((L,), 1.0, jnp.float32)
        zb_v[r, :] = jnp.zeros((L,), jnp.float32)
        return 0

    lax.fori_loop(0, CH, fill, 0)
    for k in range(NGRP):
        pltpu.sync_copy(dst_hbm.at[pl.ds((k * NW + wid) * 8, 8)],
                        idx_all.at[pl.ds(k * 8, 8)])
    for k in range(5):
        pltpu.sync_copy(
            zb_v, acc.at[pl.ds(sid * ROWS_PER_TILE + k * CH, CH)])
    plsc.subcore_barrier()

    def group(k, _):
        descs = []
        for j in range(8):
            descs.append(
                pltpu.async_copy(ones_v, acc.at[idx_all.at[k * 8 + j]], sem,
                                 add=True))
        for d in descs:
            d.wait()
        return 0

    lax.fori_loop(0, NGRP, group, 0)
    plsc.subcore_barrier()
    pltpu.sync_copy(
        acc.at[pl.ds(sid * ROWS_PER_TILE, ROWS_PER_TILE)],
        deg_hbm.at[cid, pl.ds(sid * ROWS_PER_TILE, ROWS_PER_TILE)],
    )


@functools.partial(
    pl.kernel,
    out_type=jax.ShapeDtypeStruct((NC, NPAD, D_H), jnp.float32),
    mesh=_MESH,
    scratch_types=[
        pltpu.VMEM((CH,), jnp.int32),
        pltpu.VMEM((CH,), jnp.int32),
        pltpu.VMEM((CH,), jnp.int32),
        pltpu.VMEM((CH,), jnp.int32),
        pltpu.VMEM((CH, D_H), jnp.float32),
        pltpu.VMEM((CH, D_H), jnp.float32),
        pltpu.VMEM_SHARED((NPAD, D_H), jnp.float32),
        pltpu.SemaphoreType.DMA,
        pltpu.SemaphoreType.DMA,
        pltpu.SemaphoreType.DMA,
        pltpu.SemaphoreType.DMA,
        pltpu.SemaphoreType.DMA,
        pltpu.SemaphoreType.DMA,
    ],
)
def _agg_sc(g_hbm, src_hbm, dst_hbm, out_hbm, is0, is1, id0, id1, rows0,
            rows1, acc, gs0, gs1, es0, es1, fs0, fs1):
    cid = lax.axis_index("c")
    sid = lax.axis_index("s")
    wid = _wid()
    rows = (rows0, rows1)
    iss = (is0, is1)
    ids = (id0, id1)
    gsem = (gs0, gs1)
    esem = (es0, es1)
    fsem = (fs0, fs1)

    def row(c):
        return c * NW + wid

    def fill(r, _):
        for c8 in range(D_H // L):
            rows0[r, pl.ds(c8 * L, L)] = jnp.zeros((L,), jnp.float32)
        return 0

    lax.fori_loop(0, CH, fill, 0)
    for k in range(5):
        pltpu.sync_copy(
            rows0, acc.at[pl.ds(sid * ROWS_PER_TILE + k * CH, CH)])
    plsc.subcore_barrier()

    pltpu.sync_copy(src_hbm.at[row(0)], is0)
    pltpu.sync_copy(dst_hbm.at[row(0)], id0)
    pltpu.async_copy(src_hbm.at[row(1)], is1, es1)
    pltpu.async_copy(dst_hbm.at[row(1)], id1, fs1)
    pltpu.async_copy(g_hbm.at[is0], rows0, gs0)

    def pair(p, _):
        for b in range(2):
            c = 2 * p + b
            pltpu.make_async_copy(g_hbm.at[iss[b]], rows[b], gsem[b]).wait()

            @pl.when(c + 1 < NCH_T)
            def _():
                pltpu.make_async_copy(src_hbm.at[row(c + 1)], iss[1 - b],
                                      esem[1 - b]).wait()
                pltpu.make_async_copy(dst_hbm.at[row(c + 1)], ids[1 - b],
                                      fsem[1 - b]).wait()
                pltpu.async_copy(g_hbm.at[iss[1 - b]], rows[1 - b],
                                 gsem[1 - b])

            pltpu.sync_copy(rows[b], acc.at[ids[b]], add=True)

            @pl.when(c + 2 < NCH_T)
            def _():
                pltpu.async_copy(src_hbm.at[row(c + 2)], iss[b], esem[b])
                pltpu.async_copy(dst_hbm.at[row(c + 2)], ids[b], fsem[b])
        return 0

    lax.fori_loop(0, NPAIR, pair, 0)
    plsc.subcore_barrier()
    pltpu.sync_copy(
        acc.at[pl.ds(sid * ROWS_PER_TILE, ROWS_PER_TILE)],
        out_hbm.at[cid, pl.ds(sid * ROWS_PER_TILE, ROWS_PER_TILE)],
    )


@functools.partial(
    pl.kernel,
    out_type=jax.ShapeDtypeStruct((E_P * L,), jnp.float32),
    mesh=_MESH,
    scratch_types=[
        pltpu.VMEM((NCH_T, CH), jnp.int32),
        pltpu.VMEM((NCH_T, CH), jnp.int32),
        pltpu.VMEM((CH, D_H), jnp.float32),
        pltpu.VMEM((CH, D_H), jnp.float32),
        pltpu.VMEM((CH, D_H), jnp.float32),
        pltpu.VMEM((CH, D_H), jnp.float32),
        pltpu.VMEM((CH * L,), jnp.float32),
        pltpu.VMEM((CH * L,), jnp.float32),
        pltpu.SemaphoreType.DMA,
        pltpu.SemaphoreType.DMA,
        pltpu.SemaphoreType.DMA,
        pltpu.SemaphoreType.DMA,
        pltpu.SemaphoreType.DMA,
        pltpu.SemaphoreType.DMA,
    ],
)
def _dec_sc(z_hbm, src_hbm, dst_hbm, q_hbm, idx_s, idx_d, zs0, zs1, zd0, zd1,
            q0, q1, ss0, ss1, sd0, sd1, sq0, sq1):
    wid = _wid()
    zs = (zs0, zs1)
    zd = (zd0, zd1)
    qb = (q0, q1)
    sss = (ss0, ss1)
    sds = (sd0, sd1)
    sqs = (sq0, sq1)

    for k in range(NGRP):
        pltpu.sync_copy(src_hbm.at[pl.ds((k * NW + wid) * 8, 8)],
                        idx_s.at[pl.ds(k * 8, 8)])
        pltpu.sync_copy(dst_hbm.at[pl.ds((k * NW + wid) * 8, 8)],
                        idx_d.at[pl.ds(k * 8, 8)])

    pltpu.async_copy(z_hbm.at[idx_s.at[0]], zs0, ss0)
    pltpu.async_copy(z_hbm.at[idx_d.at[0]], zd0, sd0)

    def pair(p, _):
        for b in range(2):
            c = 2 * p + b
            base = (c * NW + wid) * CH

            @pl.when(c + 1 < NCH_T)
            def _():
                pltpu.async_copy(z_hbm.at[idx_s.at[c + 1]], zs[1 - b],
                                 sss[1 - b])
                pltpu.async_copy(z_hbm.at[idx_d.at[c + 1]], zd[1 - b],
                                 sds[1 - b])

            pltpu.make_async_copy(z_hbm.at[idx_s.at[c]], zs[b], sss[b]).wait()
            pltpu.make_async_copy(z_hbm.at[idx_d.at[c]], zd[b], sds[b]).wait()

            @pl.when(c >= 2)
            def _():
                pltpu.make_async_copy(
                    qb[b], q_hbm.at[pl.ds(base * L, CH * L)], sqs[b]).wait()

            zsb = zs[b]
            zdb = zd[b]
            qvb = qb[b]

            def dot_edge(i, _):
                for u in range(2):
                    e = 2 * i + u
                    q = zsb[e, pl.ds(0, L)] * zdb[e, pl.ds(0, L)]
                    for s in range(1, D_Z // L):
                        q = q + (zsb[e, pl.ds(s * L, L)] *
                                 zdb[e, pl.ds(s * L, L)])
                    qvb[pl.ds(e * L, L)] = q
                return 0

            lax.fori_loop(0, CH // 2, dot_edge, 0)
            pltpu.async_copy(qvb, q_hbm.at[pl.ds(base * L, CH * L)], sqs[b])
        return 0

    lax.fori_loop(0, NPAIR, pair, 0)
    for b in range(2):
        c = NCH_T - 2 + b
        base = (c * NW + wid) * CH
        pltpu.make_async_copy(
            qb[b], q_hbm.at[pl.ds(base * L, CH * L)], sqs[b]).wait()


def _t1_body(x_ref, w_ref, d0_ref, d1_ref, g_ref):
    deg = d0_ref[:, 0:1] + d1_ref[:, 0:1] + 1.0
    dinv = lax.rsqrt(jnp.maximum(deg, 1e-12))
    h = jnp.dot(x_ref[...], w_ref[...], preferred_element_type=jnp.float32)
    g_ref[...] = h * dinv


def _t2_body(s0_ref, s1_ref, g1_ref, d0_ref, d1_ref, b1_ref, w_ref, g2_ref):
    deg = d0_ref[:, 0:1] + d1_ref[:, 0:1] + 1.0
    dinv = lax.rsqrt(jnp.maximum(deg, 1e-12))
    h = jnp.maximum(
        dinv * (s0_ref[...] + s1_ref[...] + g1_ref[...]) + b1_ref[...], 0.0)
    p = jnp.dot(h, w_ref[...], preferred_element_type=jnp.float32)
    g2_ref[...] = p * dinv


def _t3_body(s0_ref, s1_ref, g2_ref, d0_ref, d1_ref, bc_ref, eps_ref, z_ref):
    deg = d0_ref[:, 0:1] + d1_ref[:, 0:1] + 1.0
    dinv = lax.rsqrt(jnp.maximum(deg, 1e-12))
    o = dinv * (s0_ref[...] + s1_ref[...] + g2_ref[...]) + bc_ref[...]
    mu = o[:, :D_Z]
    lv = o[:, D_Z:]
    z = mu + jnp.exp(0.5 * lv) * eps_ref[...]
    z_ref[...] = jnp.concatenate([z, jnp.zeros_like(z)], axis=1)


def _t4_body(q_ref, o_ref):
    o_ref[...] = jax.nn.sigmoid(jnp.sum(q_ref[...], axis=1, keepdims=True))


_RB = 1000
_GRID = N // _RB


def _row_spec(width):
    return pl.BlockSpec((_RB, width), lambda i: (i, 0))


def _full_spec(shape):
    return pl.BlockSpec(shape, lambda i: tuple(0 for _ in shape))


def kernel(x, edge_index, W1, b1, W_mu, b_mu, W_lv, b_lv):
    src = edge_index[0]
    dst = edge_index[1]
    pad = E_P - E
    src_a = jnp.concatenate([src, jnp.zeros((pad,), src.dtype)])
    dst_a = jnp.concatenate([dst, jnp.full((pad,), NPAD - 1, dst.dtype)])
    dst_0 = jnp.concatenate([dst, jnp.zeros((pad,), dst.dtype)])
    src2 = src_a.reshape(NCHP, CH)
    dst2 = dst_a.reshape(NCHP, CH)
    dst2_0 = dst_0.reshape(NCHP, CH)
    Wcat = jnp.concatenate([W_mu, W_lv], axis=1)
    bcat = jnp.concatenate([b_mu, b_lv], axis=0).reshape(1, 2 * D_Z)
    b1r = b1.reshape(1, D_H)
    eps = jax.random.normal(jax.random.key(42), (N, D_Z), jnp.float32)

    deg_parts = _deg_sc(dst2)
    d0 = deg_parts[0, :N]
    d1 = deg_parts[1, :N]

    g1 = pl.pallas_call(
        _t1_body,
        grid=(_GRID,),
        in_specs=[_row_spec(D_IN), _full_spec((D_IN, D_H)), _row_spec(L),
                  _row_spec(L)],
        out_specs=_row_spec(D_H),
        out_shape=jax.ShapeDtypeStruct((N, D_H), jnp.float32),
    )(x, W1, d0, d1)

    s1p = _agg_sc(g1, src2, dst2)
    s1 = (s1p[0, :N], s1p[1, :N])

    g2 = pl.pallas_call(
        _t2_body,
        grid=(_GRID,),
        in_specs=[_row_spec(D_H), _row_spec(D_H), _row_spec(D_H),
                  _row_spec(L), _row_spec(L), _full_spec((1, D_H)),
                  _full_spec((D_H, D_H))],
        out_specs=_row_spec(D_H),
        out_shape=jax.ShapeDtypeStruct((N, D_H), jnp.float32),
    )(s1[0], s1[1], g1, d0, d1, b1r, Wcat)

    s2p = _agg_sc(g2, src2, dst2)
    s2 = (s2p[0, :N], s2p[1, :N])

    z = pl.pallas_call(
        _t3_body,
        grid=(_GRID,),
        in_specs=[_row_spec(D_H), _row_spec(D_H), _row_spec(D_H),
                  _row_spec(L), _row_spec(L), _full_spec((1, D_H)),
                  _row_spec(D_Z)],
        out_specs=_row_spec(D_H),
        out_shape=jax.ShapeDtypeStruct((N, D_H), jnp.float32),
    )(s2[0], s2[1], g2, d0, d1, bcat, eps)

    qflat = _dec_sc(z, src2, dst2_0)
    q = qflat.reshape(E_P, L)

    _EB = 4096
    out = pl.pallas_call(
        _t4_body,
        grid=(E_P // _EB,),
        in_specs=[pl.BlockSpec((_EB, L), lambda i: (i, 0))],
        out_specs=pl.BlockSpec((_EB, 1), lambda i: (i, 0)),
        out_shape=jax.ShapeDtypeStruct((E_P, 1), jnp.float32),
    )(q)
    return out[:E].reshape(E)
